# baseline (speedup 1.0000x reference)
import functools

import jax
import jax.numpy as jnp
from jax import lax
from jax.experimental import pallas as pl
from jax.experimental.pallas import tpu as pltpu
from jax.experimental.pallas import tpu_sc as plsc

N = 10000
NP = 10240
E = 160000
NTILES = 16
NCORES = 2
NB_ = NTILES * NCORES
RPT = NP // NB_
EG = 128
HG = 64
GB = 8
GCAP = 1256
NBK = GCAP // GB
CAPE = GCAP * EG
BM = 512


@functools.lru_cache(maxsize=None)
def _make_prop(C, W):
    mesh = plsc.VectorSubcoreMesh(core_axis_name="c", subcore_axis_name="s")
    NS = W // 16

    def body(hs, sidx, didx, bcnt, out, acc, ra, rb, iv, dv, bv, sga, sgb):
        cid = lax.axis_index("c")
        sid = lax.axis_index("s")
        tid = cid * NTILES + sid
        zvec = jnp.zeros((16,), jnp.float32)
        bufs = (ra, rb)
        gsems = (sga, sgb)

        pltpu.sync_copy(bcnt, bv)
        myb = bv[tid][0]

        for chunk in range(C):
            def zrow(i, carry):
                for j in range(NS):
                    acc[i, pl.ds(j * 16, 16)] = zvec
                return carry

            lax.fori_loop(0, RPT, zrow, 0)
            off = jnp.full((16,), chunk * NP, jnp.int32)

            def accumulate(buf, g, h):
                def jbody(j, cj):
                    dvec = dv[g, pl.ds(h * HG + j * 16, 16)]
                    for k in range(16):
                        e = j * 16 + k
                        dl = dvec[k]
                        for q in range(NS):
                            s = pl.ds(q * 16, 16)
                            acc[dl, s] = acc[dl, s] + buf[e, s]
                    return cj

                lax.fori_loop(0, HG // 16, jbody, 0)

            def fire(g, h, t):
                return pltpu.async_copy(
                    hs.at[iv.at[g, pl.ds(h * HG, HG)]], bufs[t], gsems[t])

            def wait(g, h, t):
                pltpu.make_async_copy(
                    hs.at[iv.at[g, pl.ds(h * HG, HG)]], bufs[t],
                    gsems[t]).wait()

            def block_body(nb, carry):
                pltpu.sync_copy(sidx.at[tid, nb], iv)
                pltpu.sync_copy(didx.at[tid, nb], dv)
                for r in range(GB):
                    for j in range(8):
                        iv[r, pl.ds(j * 16, 16)] = (
                            iv[r, pl.ds(j * 16, 16)] + off)
                fire(0, 0, 0)

                def gbody(i, c2):
                    for t in range(2):
                        g, h = divmod(i * 2 + t, 2)
                        gn, hn = divmod(i * 2 + t + 1, 2)

                        @pl.when(gn < GB)
                        def _():
                            fire(gn, hn, 1 - t)

                        wait(g, h, t)
                        accumulate(bufs[t], g, h)
                    return c2

                lax.fori_loop(0, GB, gbody, 0)
                return carry

            lax.fori_loop(0, myb, block_body, 0)

            pltpu.sync_copy(
                acc, out.at[pl.ds(chunk * NP + tid * RPT, RPT)])

    return pl.kernel(
        body,
        mesh=mesh,
        compiler_params=pltpu.CompilerParams(needs_layout_passes=False),
        out_type=jax.ShapeDtypeStruct((C * NP, W), jnp.float32),
        scratch_types=[
            pltpu.VMEM((RPT, W), jnp.float32),
            pltpu.VMEM((HG, W), jnp.float32),
            pltpu.VMEM((HG, W), jnp.float32),
            pltpu.VMEM((GB, EG), jnp.int32),
            pltpu.VMEM((GB, EG), jnp.int32),
            pltpu.VMEM((NB_, 16), jnp.int32),
            pltpu.SemaphoreType.DMA,
            pltpu.SemaphoreType.DMA,
        ],
    )


def _prop(hs3, sidx, didx, bcnt):
    C, _, W = hs3.shape
    out = _make_prop(C, W)(hs3.reshape(C * NP, W), sidx, didx, bcnt)
    return out.reshape(C, NP, W)


def _mm(x, w, b, d2, hsp, init, mode, wo):
    if mode == "raw":
        K = x.shape[1]
        wi = 256
    else:
        wi = x.shape[2]
        K = x.shape[0] * wi
    Fout = w.shape[1]
    Cin = K // wi
    BKC = 2 if (Cin % 2 == 0 and wi < 256) else 1
    KG = Cin // BKC
    Cout = Fout // wo
    w4 = w.reshape(Cin, BKC * wi, Fout) if BKC == 2 else w.reshape(Cin, wi, Fout)
    w4 = w.reshape(KG, BKC * wi, Fout)

    grid = (NP // BM, Cout, KG)

    if mode == "raw":
        x_spec = pl.BlockSpec((BM, BKC * wi), lambda i, j, k: (i, k))
    else:
        x_spec = pl.BlockSpec((BKC, BM, wi), lambda i, j, k: (k, i, 0))
    w_spec = pl.BlockSpec((1, BKC * wi, wo), lambda i, j, k: (k, 0, j))
    d_spec = pl.BlockSpec((BM, 128), lambda i, j, k: (i, 0))
    io_spec = pl.BlockSpec((1, BM, wo), lambda i, j, k: (j, i, 0))

    in_specs = [x_spec, w_spec, d_spec]
    args = [x, w4, d2]
    if mode == "mid":
        in_specs += [x_spec,
                     pl.BlockSpec((BKC, 1, wi), lambda i, j, k: (k, 0, 0))]
        args += [hsp, b.reshape(Cin, 1, wi)]
    if init is not None:
        in_specs.append(io_spec)
        args.append(init)

    def body(*refs):
        if mode == "mid" and init is not None:
            x_ref, w_ref, d_ref, h_ref, b_ref, i_ref, o_ref, acc = refs
        elif mode == "mid":
            x_ref, w_ref, d_ref, h_ref, b_ref, o_ref, acc = refs
            i_ref = None
        elif init is not None:
            x_ref, w_ref, d_ref, i_ref, o_ref, acc = refs
        else:
            x_ref, w_ref, d_ref, o_ref, acc = refs
            i_ref = None
        k = pl.program_id(2)

        @pl.when(k == 0)
        def _():
            acc[...] = jnp.zeros((BM, wo), jnp.float32)

        d1 = d_ref[:, :1]
        if mode == "mid":
            xs = [jnp.maximum(d1 * (x_ref[t] + h_ref[t])
                              + b_ref[t, 0][None, :], 0.0)
                  for t in range(BKC)]
            xb = xs[0] if BKC == 1 else jnp.concatenate(xs, axis=1)
        else:
            xb = x_ref[...]
        acc[...] += jnp.dot(xb, w_ref[0], preferred_element_type=jnp.float32)

        @pl.when(k == KG - 1)
        def _():
            r = d1 * acc[...]
            if i_ref is not None:
                r = r + i_ref[0]
            o_ref[0] = r

    return pl.pallas_call(
        body,
        grid=grid,
        in_specs=in_specs,
        out_specs=io_spec,
        out_shape=jax.ShapeDtypeStruct((Cout, NP, wo), jnp.float32),
        scratch_shapes=[pltpu.VMEM((BM, wo), jnp.float32)],
        compiler_params=pltpu.CompilerParams(
            dimension_semantics=("parallel", "parallel", "arbitrary")),
    )(*args)


def _elemwise(body, out_shape, *arrays):
    spec = pl.BlockSpec((BM, 128), lambda i: (i, 0))
    return pl.pallas_call(
        body,
        grid=(NP // BM,),
        in_specs=[spec] * len(arrays),
        out_specs=spec,
        out_shape=out_shape,
    )(*arrays)


def _dinv2(sdeg, mask2):
    def body(s_ref, m_ref, d_ref):
        d_ref[...] = m_ref[...] * lax.rsqrt(s_ref[...] + 1.0)

    return _elemwise(body, jax.ShapeDtypeStruct((NP, 128), jnp.float32),
                     sdeg, mask2)


def _finalize(s, hs, b2, d2):
    bfull = jnp.broadcast_to(b2[None, :], (NP, 128))

    def body(s_ref, h_ref, b_ref, d_ref, o_ref):
        o_ref[...] = d_ref[...] * (s_ref[...] + h_ref[...]) + b_ref[...]

    return _elemwise(body, jax.ShapeDtypeStruct((NP, 128), jnp.float32),
                     s, hs, bfull, d2)


def _pad_w(w, rows, cols):
    return jnp.pad(w, ((0, rows - w.shape[0]), (0, cols - w.shape[1])))


def kernel(vertices, feats1, feats2, feats3, edge_index, params):
    f32 = jnp.float32
    src = edge_index[0].astype(jnp.int32)
    dst = edge_index[1].astype(jnp.int32)
    bucket = dst // RPT
    oh = (bucket[:, None] == jnp.arange(NB_, dtype=jnp.int32)[None, :])
    rank = jnp.cumsum(oh.astype(jnp.int32), axis=0) - oh.astype(jnp.int32)
    rank = jnp.sum(rank * oh, axis=1)
    cnt = jnp.sum(oh, axis=0)
    pos = bucket * CAPE + rank
    src_blk = jnp.full((NB_ * CAPE,), NP - 1, jnp.int32).at[pos].set(src)
    dstl_blk = jnp.zeros((NB_ * CAPE,), jnp.int32).at[pos].set(dst - bucket * RPT)
    sidx = src_blk.reshape(NB_, NBK, GB, EG)
    didx = dstl_blk.reshape(NB_, NBK, GB, EG)
    bcnt = ((cnt + (GB * EG - 1)) // (GB * EG)).astype(jnp.int32)
    bcnt = jnp.broadcast_to(bcnt[:, None], (NB_, 16))

    mask2 = jnp.broadcast_to(
        (jnp.arange(NP) < N).astype(f32)[:, None], (NP, 128))
    sdeg = _prop(mask2[None], sidx, didx, bcnt)[0]
    d2 = _dinv2(sdeg, mask2)

    p1, p2, p3 = params["block1"], params["block2"], params["block3"]

    def chain_rest(hs0, p):
        s0 = _prop(hs0, sidx, didx, bcnt)
        hs1 = _mm(s0, p["W1"], p["b0"], d2, hs0, None, "mid", 256)
        s1 = _prop(hs1, sidx, didx, bcnt)
        hs2 = _mm(s1, p["W2"], p["b1"], d2, hs1, None, "mid", 256)
        s2 = _prop(hs2, sidx, didx, bcnt)
        hs3 = _mm(s2, p["W3"], p["b2"], d2, hs2, None, "mid", 128)
        s3 = _prop(hs3, sidx, didx, bcnt)
        hs4 = _mm(s3, _pad_w(p["W4"], 128, 128), p["b3"], d2, hs3, None,
                  "mid", 128)
        s4 = _prop(hs4, sidx, didx, bcnt)
        b4p = jnp.pad(p["b4"], (0, 128 - 3))
        coord = _finalize(s4[0], hs4[0], b4p, d2)[:N, :3]
        return s3, hs3, coord

    x0 = jnp.concatenate([feats1, vertices], axis=1)
    x0 = jnp.pad(x0, ((0, NP - N), (0, 1536 - 1283)))
    hs0 = _mm(x0, _pad_w(p1["W0"], 1536, 1024), None, d2, None, None,
              "raw", 256)
    s3_1, hs3_1, coord_1 = chain_rest(hs0, p1)

    pinit = _mm(s3_1, p2["W0"][1280:, :], p1["b3"], d2, hs3_1, None,
                "mid", 256)
    f2p = jnp.pad(feats2, ((0, NP - N), (0, 0)))
    hs0 = _mm(f2p, p2["W0"][:1280, :], None, d2, None, pinit, "raw", 256)
    s3_2, hs3_2, coord_2 = chain_rest(hs0, p2)

    pinit = _mm(s3_2, p3["W0"][1280:, :], p2["b3"], d2, hs3_2, None,
                "mid", 256)
    f3p = jnp.pad(feats3, ((0, NP - N), (0, 0)))
    hs0 = _mm(f3p, p3["W0"][:1280, :], None, d2, None, pinit, "raw", 256)
    _, _, coord_3 = chain_rest(hs0, p3)

    return (vertices, coord_1, coord_1, coord_2, coord_2, coord_3)


# speedup vs baseline: 1.2725x; 1.2725x over previous
import functools

import jax
import jax.numpy as jnp
from jax import lax
from jax.experimental import pallas as pl
from jax.experimental.pallas import tpu as pltpu
from jax.experimental.pallas import tpu_sc as plsc

N = 10000
NP = 10240
E = 160000
NTILES = 16
NCORES = 2
NB_ = NTILES * NCORES
RPT = NP // NB_
EG = 128
HG = 64
GB = 8
GCAP = 1256
NBK = GCAP // GB
CAPE = GCAP * EG
BM = 512


@functools.lru_cache(maxsize=None)
def _make_prop(C, W):
    mesh = plsc.VectorSubcoreMesh(core_axis_name="c", subcore_axis_name="s")
    NS = W // 16

    def body(hs, sidx, didx, bcnt, out, acc, ra, rb, iv, dv, bv, sga, sgb):
        cid = lax.axis_index("c")
        sid = lax.axis_index("s")
        tid = cid * NTILES + sid
        zvec = jnp.zeros((16,), jnp.float32)
        bufs = (ra, rb)
        gsems = (sga, sgb)

        pltpu.sync_copy(bcnt, bv)
        myb = bv[tid][0]

        for chunk in range(C):
            def zrow(i, carry):
                for j in range(NS):
                    acc[i, pl.ds(j * 16, 16)] = zvec
                return carry

            lax.fori_loop(0, RPT, zrow, 0)
            off = jnp.full((16,), chunk * NP, jnp.int32)

            def accumulate(buf, g, h):
                def jbody(j, cj):
                    dvec = dv[g, pl.ds(h * HG + j * 16, 16)]
                    for k in range(16):
                        e = j * 16 + k
                        dl = dvec[k]

                        @plsc.parallel_loop(0, NS, unroll=NS)
                        def _(q):
                            s = pl.ds(q * 16, 16)
                            acc[dl, s] = acc[dl, s] + buf[e, s]
                    return cj

                lax.fori_loop(0, HG // 16, jbody, 0)

            def fire(g, h, t):
                return pltpu.async_copy(
                    hs.at[iv.at[g, pl.ds(h * HG, HG)]], bufs[t], gsems[t])

            def wait(g, h, t):
                pltpu.make_async_copy(
                    hs.at[iv.at[g, pl.ds(h * HG, HG)]], bufs[t],
                    gsems[t]).wait()

            def block_body(nb, carry):
                pltpu.sync_copy(sidx.at[tid, nb], iv)
                pltpu.sync_copy(didx.at[tid, nb], dv)
                for r in range(GB):
                    for j in range(8):
                        iv[r, pl.ds(j * 16, 16)] = (
                            iv[r, pl.ds(j * 16, 16)] + off)
                fire(0, 0, 0)

                def gbody(i, c2):
                    for t in range(2):
                        g, h = divmod(i * 2 + t, 2)
                        gn, hn = divmod(i * 2 + t + 1, 2)

                        @pl.when(gn < GB)
                        def _():
                            fire(gn, hn, 1 - t)

                        wait(g, h, t)
                        accumulate(bufs[t], g, h)
                    return c2

                lax.fori_loop(0, GB, gbody, 0)
                return carry

            lax.fori_loop(0, myb, block_body, 0)

            pltpu.sync_copy(
                acc, out.at[pl.ds(chunk * NP + tid * RPT, RPT)])

    return pl.kernel(
        body,
        mesh=mesh,
        compiler_params=pltpu.CompilerParams(needs_layout_passes=False),
        out_type=jax.ShapeDtypeStruct((C * NP, W), jnp.float32),
        scratch_types=[
            pltpu.VMEM((RPT, W), jnp.float32),
            pltpu.VMEM((HG, W), jnp.float32),
            pltpu.VMEM((HG, W), jnp.float32),
            pltpu.VMEM((GB, EG), jnp.int32),
            pltpu.VMEM((GB, EG), jnp.int32),
            pltpu.VMEM((NB_, 16), jnp.int32),
            pltpu.SemaphoreType.DMA,
            pltpu.SemaphoreType.DMA,
        ],
    )


def _prop(hs3, sidx, didx, bcnt):
    C, _, W = hs3.shape
    out = _make_prop(C, W)(hs3.reshape(C * NP, W), sidx, didx, bcnt)
    return out.reshape(C, NP, W)


def _mm(x, w, b, d2, hsp, init, mode, wo):
    if mode == "raw":
        K = x.shape[1]
        wi = 256
    else:
        wi = x.shape[2]
        K = x.shape[0] * wi
    Fout = w.shape[1]
    Cin = K // wi
    BKC = 2 if (Cin % 2 == 0 and wi < 256) else 1
    KG = Cin // BKC
    Cout = Fout // wo
    w4 = w.reshape(Cin, BKC * wi, Fout) if BKC == 2 else w.reshape(Cin, wi, Fout)
    w4 = w.reshape(KG, BKC * wi, Fout)

    grid = (NP // BM, Cout, KG)

    if mode == "raw":
        x_spec = pl.BlockSpec((BM, BKC * wi), lambda i, j, k: (i, k))
    else:
        x_spec = pl.BlockSpec((BKC, BM, wi), lambda i, j, k: (k, i, 0))
    w_spec = pl.BlockSpec((1, BKC * wi, wo), lambda i, j, k: (k, 0, j))
    d_spec = pl.BlockSpec((BM, 128), lambda i, j, k: (i, 0))
    io_spec = pl.BlockSpec((1, BM, wo), lambda i, j, k: (j, i, 0))

    in_specs = [x_spec, w_spec, d_spec]
    args = [x, w4, d2]
    if mode == "mid":
        in_specs += [x_spec,
                     pl.BlockSpec((BKC, 1, wi), lambda i, j, k: (k, 0, 0))]
        args += [hsp, b.reshape(Cin, 1, wi)]
    if init is not None:
        in_specs.append(io_spec)
        args.append(init)

    def body(*refs):
        if mode == "mid" and init is not None:
            x_ref, w_ref, d_ref, h_ref, b_ref, i_ref, o_ref, acc = refs
        elif mode == "mid":
            x_ref, w_ref, d_ref, h_ref, b_ref, o_ref, acc = refs
            i_ref = None
        elif init is not None:
            x_ref, w_ref, d_ref, i_ref, o_ref, acc = refs
        else:
            x_ref, w_ref, d_ref, o_ref, acc = refs
            i_ref = None
        k = pl.program_id(2)

        @pl.when(k == 0)
        def _():
            acc[...] = jnp.zeros((BM, wo), jnp.float32)

        d1 = d_ref[:, :1]
        if mode == "mid":
            xs = [jnp.maximum(d1 * (x_ref[t] + h_ref[t])
                              + b_ref[t, 0][None, :], 0.0)
                  for t in range(BKC)]
            xb = xs[0] if BKC == 1 else jnp.concatenate(xs, axis=1)
        else:
            xb = x_ref[...]
        acc[...] += jnp.dot(xb, w_ref[0], preferred_element_type=jnp.float32)

        @pl.when(k == KG - 1)
        def _():
            r = d1 * acc[...]
            if i_ref is not None:
                r = r + i_ref[0]
            o_ref[0] = r

    return pl.pallas_call(
        body,
        grid=grid,
        in_specs=in_specs,
        out_specs=io_spec,
        out_shape=jax.ShapeDtypeStruct((Cout, NP, wo), jnp.float32),
        scratch_shapes=[pltpu.VMEM((BM, wo), jnp.float32)],
        compiler_params=pltpu.CompilerParams(
            dimension_semantics=("parallel", "parallel", "arbitrary")),
    )(*args)


def _elemwise(body, out_shape, *arrays):
    spec = pl.BlockSpec((BM, 128), lambda i: (i, 0))
    return pl.pallas_call(
        body,
        grid=(NP // BM,),
        in_specs=[spec] * len(arrays),
        out_specs=spec,
        out_shape=out_shape,
    )(*arrays)


def _dinv2(sdeg, mask2):
    def body(s_ref, m_ref, d_ref):
        d_ref[...] = m_ref[...] * lax.rsqrt(s_ref[...] + 1.0)

    return _elemwise(body, jax.ShapeDtypeStruct((NP, 128), jnp.float32),
                     sdeg, mask2)


def _finalize(s, hs, b2, d2):
    bfull = jnp.broadcast_to(b2[None, :], (NP, 128))

    def body(s_ref, h_ref, b_ref, d_ref, o_ref):
        o_ref[...] = d_ref[...] * (s_ref[...] + h_ref[...]) + b_ref[...]

    return _elemwise(body, jax.ShapeDtypeStruct((NP, 128), jnp.float32),
                     s, hs, bfull, d2)


def _pad_w(w, rows, cols):
    return jnp.pad(w, ((0, rows - w.shape[0]), (0, cols - w.shape[1])))


def kernel(vertices, feats1, feats2, feats3, edge_index, params):
    f32 = jnp.float32
    src = edge_index[0].astype(jnp.int32)
    dst = edge_index[1].astype(jnp.int32)
    bucket = dst // RPT
    oh = (bucket[:, None] == jnp.arange(NB_, dtype=jnp.int32)[None, :])
    rank = jnp.cumsum(oh.astype(jnp.int32), axis=0) - oh.astype(jnp.int32)
    rank = jnp.sum(rank * oh, axis=1)
    cnt = jnp.sum(oh, axis=0)
    pos = bucket * CAPE + rank
    src_blk = jnp.full((NB_ * CAPE,), NP - 1, jnp.int32).at[pos].set(src)
    dstl_blk = jnp.zeros((NB_ * CAPE,), jnp.int32).at[pos].set(dst - bucket * RPT)
    sidx = src_blk.reshape(NB_, NBK, GB, EG)
    didx = dstl_blk.reshape(NB_, NBK, GB, EG)
    bcnt = ((cnt + (GB * EG - 1)) // (GB * EG)).astype(jnp.int32)
    bcnt = jnp.broadcast_to(bcnt[:, None], (NB_, 16))

    mask2 = jnp.broadcast_to(
        (jnp.arange(NP) < N).astype(f32)[:, None], (NP, 128))
    sdeg = _prop(mask2[None], sidx, didx, bcnt)[0]
    d2 = _dinv2(sdeg, mask2)

    p1, p2, p3 = params["block1"], params["block2"], params["block3"]

    def chain_rest(hs0, p):
        s0 = _prop(hs0, sidx, didx, bcnt)
        hs1 = _mm(s0, p["W1"], p["b0"], d2, hs0, None, "mid", 256)
        s1 = _prop(hs1, sidx, didx, bcnt)
        hs2 = _mm(s1, p["W2"], p["b1"], d2, hs1, None, "mid", 256)
        s2 = _prop(hs2, sidx, didx, bcnt)
        hs3 = _mm(s2, p["W3"], p["b2"], d2, hs2, None, "mid", 128)
        s3 = _prop(hs3, sidx, didx, bcnt)
        hs4 = _mm(s3, _pad_w(p["W4"], 128, 128), p["b3"], d2, hs3, None,
                  "mid", 128)
        s4 = _prop(hs4, sidx, didx, bcnt)
        b4p = jnp.pad(p["b4"], (0, 128 - 3))
        coord = _finalize(s4[0], hs4[0], b4p, d2)[:N, :3]
        return s3, hs3, coord

    x0 = jnp.concatenate([feats1, vertices], axis=1)
    x0 = jnp.pad(x0, ((0, NP - N), (0, 1536 - 1283)))
    hs0 = _mm(x0, _pad_w(p1["W0"], 1536, 1024), None, d2, None, None,
              "raw", 256)
    s3_1, hs3_1, coord_1 = chain_rest(hs0, p1)

    pinit = _mm(s3_1, p2["W0"][1280:, :], p1["b3"], d2, hs3_1, None,
                "mid", 256)
    f2p = jnp.pad(feats2, ((0, NP - N), (0, 0)))
    hs0 = _mm(f2p, p2["W0"][:1280, :], None, d2, None, pinit, "raw", 256)
    s3_2, hs3_2, coord_2 = chain_rest(hs0, p2)

    pinit = _mm(s3_2, p3["W0"][1280:, :], p2["b3"], d2, hs3_2, None,
                "mid", 256)
    f3p = jnp.pad(feats3, ((0, NP - N), (0, 0)))
    hs0 = _mm(f3p, p3["W0"][:1280, :], None, d2, None, pinit, "raw", 256)
    _, _, coord_3 = chain_rest(hs0, p3)

    return (vertices, coord_1, coord_1, coord_2, coord_2, coord_3)


# speedup vs baseline: 1.2900x; 1.0138x over previous
import functools

import jax
import jax.numpy as jnp
from jax import lax
from jax.experimental import pallas as pl
from jax.experimental.pallas import tpu as pltpu
from jax.experimental.pallas import tpu_sc as plsc

N = 10000
NP = 10240
E = 160000
NTILES = 16
NCORES = 2
NB_ = NTILES * NCORES
RPT = NP // NB_
EG = 128
HG = 64
GB = 8
GCAP = 1256
NBK = GCAP // GB
CAPE = GCAP * EG
BM = 512


@functools.lru_cache(maxsize=None)
def _make_prop(C, W):
    mesh = plsc.VectorSubcoreMesh(core_axis_name="c", subcore_axis_name="s")
    NS = W // 16

    def body(hs, sidx, didx, bcnt, out, acc, ra, rb, iv, dv, bv, sga, sgb):
        cid = lax.axis_index("c")
        sid = lax.axis_index("s")
        tid = cid * NTILES + sid
        zvec = jnp.zeros((16,), jnp.float32)
        bufs = (ra, rb)
        gsems = (sga, sgb)

        pltpu.sync_copy(bcnt, bv)
        myb = bv[tid][0]

        for chunk in range(C):
            def zrow(i, carry):
                for j in range(NS):
                    acc[i, pl.ds(j * 16, 16)] = zvec
                return carry

            lax.fori_loop(0, RPT, zrow, 0)
            off = jnp.full((16,), chunk * NP, jnp.int32)

            def accumulate(buf, g, h):
                lane = lax.iota(jnp.int32, 16)
                zi = jnp.zeros((16,), jnp.int32)

                def jbody(j, cj):
                    dvec = dv[g, pl.ds(h * HG + j * 16, 16)]
                    for k in range(16):
                        e = j * 16 + k
                        rowv = zi + dvec[k]

                        @plsc.parallel_loop(0, NS, unroll=NS)
                        def _(q):
                            s = pl.ds(q * 16, 16)
                            plsc.addupdate_scatter(
                                acc, [rowv, lane + q * 16], buf[e, s])
                    return cj

                lax.fori_loop(0, HG // 16, jbody, 0)

            def fire(g, h, t):
                return pltpu.async_copy(
                    hs.at[iv.at[g, pl.ds(h * HG, HG)]], bufs[t], gsems[t])

            def wait(g, h, t):
                pltpu.make_async_copy(
                    hs.at[iv.at[g, pl.ds(h * HG, HG)]], bufs[t],
                    gsems[t]).wait()

            def block_body(nb, carry):
                pltpu.sync_copy(sidx.at[tid, nb], iv)
                pltpu.sync_copy(didx.at[tid, nb], dv)
                for r in range(GB):
                    for j in range(8):
                        iv[r, pl.ds(j * 16, 16)] = (
                            iv[r, pl.ds(j * 16, 16)] + off)
                fire(0, 0, 0)

                def gbody(i, c2):
                    for t in range(2):
                        g, h = divmod(i * 2 + t, 2)
                        gn, hn = divmod(i * 2 + t + 1, 2)

                        @pl.when(gn < GB)
                        def _():
                            fire(gn, hn, 1 - t)

                        wait(g, h, t)
                        accumulate(bufs[t], g, h)
                    return c2

                lax.fori_loop(0, GB, gbody, 0)
                return carry

            lax.fori_loop(0, myb, block_body, 0)

            pltpu.sync_copy(
                acc, out.at[pl.ds(chunk * NP + tid * RPT, RPT)])

    return pl.kernel(
        body,
        mesh=mesh,
        compiler_params=pltpu.CompilerParams(needs_layout_passes=False),
        out_type=jax.ShapeDtypeStruct((C * NP, W), jnp.float32),
        scratch_types=[
            pltpu.VMEM((RPT, W), jnp.float32),
            pltpu.VMEM((HG, W), jnp.float32),
            pltpu.VMEM((HG, W), jnp.float32),
            pltpu.VMEM((GB, EG), jnp.int32),
            pltpu.VMEM((GB, EG), jnp.int32),
            pltpu.VMEM((NB_, 16), jnp.int32),
            pltpu.SemaphoreType.DMA,
            pltpu.SemaphoreType.DMA,
        ],
    )


def _prop(hs3, sidx, didx, bcnt):
    C, _, W = hs3.shape
    out = _make_prop(C, W)(hs3.reshape(C * NP, W), sidx, didx, bcnt)
    return out.reshape(C, NP, W)


def _mm(x, w, b, d2, hsp, init, mode, wo):
    if mode == "raw":
        K = x.shape[1]
        wi = 256
    else:
        wi = x.shape[2]
        K = x.shape[0] * wi
    Fout = w.shape[1]
    Cin = K // wi
    BKC = 2 if (Cin % 2 == 0 and wi < 256) else 1
    KG = Cin // BKC
    Cout = Fout // wo
    w4 = w.reshape(Cin, BKC * wi, Fout) if BKC == 2 else w.reshape(Cin, wi, Fout)
    w4 = w.reshape(KG, BKC * wi, Fout)

    grid = (NP // BM, Cout, KG)

    if mode == "raw":
        x_spec = pl.BlockSpec((BM, BKC * wi), lambda i, j, k: (i, k))
    else:
        x_spec = pl.BlockSpec((BKC, BM, wi), lambda i, j, k: (k, i, 0))
    w_spec = pl.BlockSpec((1, BKC * wi, wo), lambda i, j, k: (k, 0, j))
    d_spec = pl.BlockSpec((BM, 128), lambda i, j, k: (i, 0))
    io_spec = pl.BlockSpec((1, BM, wo), lambda i, j, k: (j, i, 0))

    in_specs = [x_spec, w_spec, d_spec]
    args = [x, w4, d2]
    if mode == "mid":
        in_specs += [x_spec,
                     pl.BlockSpec((BKC, 1, wi), lambda i, j, k: (k, 0, 0))]
        args += [hsp, b.reshape(Cin, 1, wi)]
    if init is not None:
        in_specs.append(io_spec)
        args.append(init)

    def body(*refs):
        if mode == "mid" and init is not None:
            x_ref, w_ref, d_ref, h_ref, b_ref, i_ref, o_ref, acc = refs
        elif mode == "mid":
            x_ref, w_ref, d_ref, h_ref, b_ref, o_ref, acc = refs
            i_ref = None
        elif init is not None:
            x_ref, w_ref, d_ref, i_ref, o_ref, acc = refs
        else:
            x_ref, w_ref, d_ref, o_ref, acc = refs
            i_ref = None
        k = pl.program_id(2)

        @pl.when(k == 0)
        def _():
            acc[...] = jnp.zeros((BM, wo), jnp.float32)

        d1 = d_ref[:, :1]
        if mode == "mid":
            xs = [jnp.maximum(d1 * (x_ref[t] + h_ref[t])
                              + b_ref[t, 0][None, :], 0.0)
                  for t in range(BKC)]
            xb = xs[0] if BKC == 1 else jnp.concatenate(xs, axis=1)
        else:
            xb = x_ref[...]
        acc[...] += jnp.dot(xb, w_ref[0], preferred_element_type=jnp.float32)

        @pl.when(k == KG - 1)
        def _():
            r = d1 * acc[...]
            if i_ref is not None:
                r = r + i_ref[0]
            o_ref[0] = r

    return pl.pallas_call(
        body,
        grid=grid,
        in_specs=in_specs,
        out_specs=io_spec,
        out_shape=jax.ShapeDtypeStruct((Cout, NP, wo), jnp.float32),
        scratch_shapes=[pltpu.VMEM((BM, wo), jnp.float32)],
        compiler_params=pltpu.CompilerParams(
            dimension_semantics=("parallel", "parallel", "arbitrary")),
    )(*args)


def _elemwise(body, out_shape, *arrays):
    spec = pl.BlockSpec((BM, 128), lambda i: (i, 0))
    return pl.pallas_call(
        body,
        grid=(NP // BM,),
        in_specs=[spec] * len(arrays),
        out_specs=spec,
        out_shape=out_shape,
    )(*arrays)


def _dinv2(sdeg, mask2):
    def body(s_ref, m_ref, d_ref):
        d_ref[...] = m_ref[...] * lax.rsqrt(s_ref[...] + 1.0)

    return _elemwise(body, jax.ShapeDtypeStruct((NP, 128), jnp.float32),
                     sdeg, mask2)


def _finalize(s, hs, b2, d2):
    bfull = jnp.broadcast_to(b2[None, :], (NP, 128))

    def body(s_ref, h_ref, b_ref, d_ref, o_ref):
        o_ref[...] = d_ref[...] * (s_ref[...] + h_ref[...]) + b_ref[...]

    return _elemwise(body, jax.ShapeDtypeStruct((NP, 128), jnp.float32),
                     s, hs, bfull, d2)


def _pad_w(w, rows, cols):
    return jnp.pad(w, ((0, rows - w.shape[0]), (0, cols - w.shape[1])))


def kernel(vertices, feats1, feats2, feats3, edge_index, params):
    f32 = jnp.float32
    src = edge_index[0].astype(jnp.int32)
    dst = edge_index[1].astype(jnp.int32)
    bucket = dst // RPT
    oh = (bucket[:, None] == jnp.arange(NB_, dtype=jnp.int32)[None, :])
    rank = jnp.cumsum(oh.astype(jnp.int32), axis=0) - oh.astype(jnp.int32)
    rank = jnp.sum(rank * oh, axis=1)
    cnt = jnp.sum(oh, axis=0)
    pos = bucket * CAPE + rank
    src_blk = jnp.full((NB_ * CAPE,), NP - 1, jnp.int32).at[pos].set(src)
    dstl_blk = jnp.zeros((NB_ * CAPE,), jnp.int32).at[pos].set(dst - bucket * RPT)
    sidx = src_blk.reshape(NB_, NBK, GB, EG)
    didx = dstl_blk.reshape(NB_, NBK, GB, EG)
    bcnt = ((cnt + (GB * EG - 1)) // (GB * EG)).astype(jnp.int32)
    bcnt = jnp.broadcast_to(bcnt[:, None], (NB_, 16))

    mask2 = jnp.broadcast_to(
        (jnp.arange(NP) < N).astype(f32)[:, None], (NP, 128))
    sdeg = _prop(mask2[None], sidx, didx, bcnt)[0]
    d2 = _dinv2(sdeg, mask2)

    p1, p2, p3 = params["block1"], params["block2"], params["block3"]

    def chain_rest(hs0, p):
        s0 = _prop(hs0, sidx, didx, bcnt)
        hs1 = _mm(s0, p["W1"], p["b0"], d2, hs0, None, "mid", 256)
        s1 = _prop(hs1, sidx, didx, bcnt)
        hs2 = _mm(s1, p["W2"], p["b1"], d2, hs1, None, "mid", 256)
        s2 = _prop(hs2, sidx, didx, bcnt)
        hs3 = _mm(s2, p["W3"], p["b2"], d2, hs2, None, "mid", 128)
        s3 = _prop(hs3, sidx, didx, bcnt)
        hs4 = _mm(s3, _pad_w(p["W4"], 128, 128), p["b3"], d2, hs3, None,
                  "mid", 128)
        s4 = _prop(hs4, sidx, didx, bcnt)
        b4p = jnp.pad(p["b4"], (0, 128 - 3))
        coord = _finalize(s4[0], hs4[0], b4p, d2)[:N, :3]
        return s3, hs3, coord

    x0 = jnp.concatenate([feats1, vertices], axis=1)
    x0 = jnp.pad(x0, ((0, NP - N), (0, 1536 - 1283)))
    hs0 = _mm(x0, _pad_w(p1["W0"], 1536, 1024), None, d2, None, None,
              "raw", 256)
    s3_1, hs3_1, coord_1 = chain_rest(hs0, p1)

    pinit = _mm(s3_1, p2["W0"][1280:, :], p1["b3"], d2, hs3_1, None,
                "mid", 256)
    f2p = jnp.pad(feats2, ((0, NP - N), (0, 0)))
    hs0 = _mm(f2p, p2["W0"][:1280, :], None, d2, None, pinit, "raw", 256)
    s3_2, hs3_2, coord_2 = chain_rest(hs0, p2)

    pinit = _mm(s3_2, p3["W0"][1280:, :], p2["b3"], d2, hs3_2, None,
                "mid", 256)
    f3p = jnp.pad(feats3, ((0, NP - N), (0, 0)))
    hs0 = _mm(f3p, p3["W0"][:1280, :], None, d2, None, pinit, "raw", 256)
    _, _, coord_3 = chain_rest(hs0, p3)

    return (vertices, coord_1, coord_1, coord_2, coord_2, coord_3)


# speedup vs baseline: 1.2905x; 1.0004x over previous
import functools

import jax
import jax.numpy as jnp
from jax import lax
from jax.experimental import pallas as pl
from jax.experimental.pallas import tpu as pltpu
from jax.experimental.pallas import tpu_sc as plsc

N = 10000
NP = 10240
E = 160000
NTILES = 16
NCORES = 2
NB_ = NTILES * NCORES
RPT = NP // NB_
EG = 128
HG = 64
GB = 8
GCAP = 1256
NBK = GCAP // GB
CAPE = GCAP * EG
BM = 512


@functools.lru_cache(maxsize=None)
def _make_prop(C, W):
    mesh = plsc.VectorSubcoreMesh(core_axis_name="c", subcore_axis_name="s")
    NS = W // 16

    def body(hs, sidx, didx, bcnt, out, acc, ra, rb, iv, dv, bv, sga, sgb):
        cid = lax.axis_index("c")
        sid = lax.axis_index("s")
        tid = cid * NTILES + sid
        zvec = jnp.zeros((16,), jnp.float32)
        bufs = (ra, rb)
        gsems = (sga, sgb)

        pltpu.sync_copy(bcnt, bv)
        myb = bv[tid][0]

        for chunk in range(C):
            def zrow(i, carry):
                for j in range(NS):
                    acc[i, pl.ds(j * 16, 16)] = zvec
                return carry

            lax.fori_loop(0, RPT, zrow, 0)
            off = jnp.full((16,), chunk * NP, jnp.int32)

            def accumulate(buf, g, h):
                lane = lax.iota(jnp.int32, 16)
                zi = jnp.zeros((16,), jnp.int32)

                def jbody(j, cj):
                    dvec = dv[g, pl.ds(h * HG + j * 16, 16)]
                    for k in range(16):
                        rowv = zi + dvec[k]

                        @plsc.parallel_loop(0, NS, unroll=NS)
                        def _(q, k=k):
                            s = pl.ds(q * 16, 16)
                            plsc.addupdate_scatter(
                                acc, [rowv, lane + q * 16],
                                buf[j * 16 + k, s])
                    return cj

                lax.fori_loop(0, HG // 16, jbody, 0)

            def fire(g, h, t):
                return pltpu.async_copy(
                    hs.at[iv.at[g, pl.ds(h * HG, HG)]], bufs[t], gsems[t])

            def wait(g, h, t):
                pltpu.make_async_copy(
                    hs.at[iv.at[g, pl.ds(h * HG, HG)]], bufs[t],
                    gsems[t]).wait()

            def block_body(nb, carry):
                pltpu.sync_copy(sidx.at[tid, nb], iv)
                pltpu.sync_copy(didx.at[tid, nb], dv)
                for r in range(GB):
                    for j in range(8):
                        iv[r, pl.ds(j * 16, 16)] = (
                            iv[r, pl.ds(j * 16, 16)] + off)
                fire(0, 0, 0)

                def gbody(i, c2):
                    for t in range(2):
                        g, h = divmod(i * 2 + t, 2)
                        gn, hn = divmod(i * 2 + t + 1, 2)

                        @pl.when(gn < GB)
                        def _():
                            fire(gn, hn, 1 - t)

                        wait(g, h, t)
                        accumulate(bufs[t], g, h)
                    return c2

                lax.fori_loop(0, GB, gbody, 0)
                return carry

            lax.fori_loop(0, myb, block_body, 0)

            pltpu.sync_copy(
                acc, out.at[pl.ds(chunk * NP + tid * RPT, RPT)])

    return pl.kernel(
        body,
        mesh=mesh,
        compiler_params=pltpu.CompilerParams(needs_layout_passes=False),
        out_type=jax.ShapeDtypeStruct((C * NP, W), jnp.float32),
        scratch_types=[
            pltpu.VMEM((RPT, W), jnp.float32),
            pltpu.VMEM((HG, W), jnp.float32),
            pltpu.VMEM((HG, W), jnp.float32),
            pltpu.VMEM((GB, EG), jnp.int32),
            pltpu.VMEM((GB, EG), jnp.int32),
            pltpu.VMEM((NB_, 16), jnp.int32),
            pltpu.SemaphoreType.DMA,
            pltpu.SemaphoreType.DMA,
        ],
    )


def _prop(hs3, sidx, didx, bcnt):
    C, _, W = hs3.shape
    out = _make_prop(C, W)(hs3.reshape(C * NP, W), sidx, didx, bcnt)
    return out.reshape(C, NP, W)


def _mm(x, w, b, d2, hsp, init, mode, wo):
    if mode == "raw":
        K = x.shape[1]
        wi = 256
    else:
        wi = x.shape[2]
        K = x.shape[0] * wi
    Fout = w.shape[1]
    Cin = K // wi
    BKC = 2 if (Cin % 2 == 0 and wi < 256) else 1
    KG = Cin // BKC
    Cout = Fout // wo
    w4 = w.reshape(Cin, BKC * wi, Fout) if BKC == 2 else w.reshape(Cin, wi, Fout)
    w4 = w.reshape(KG, BKC * wi, Fout)

    grid = (NP // BM, Cout, KG)

    if mode == "raw":
        x_spec = pl.BlockSpec((BM, BKC * wi), lambda i, j, k: (i, k))
    else:
        x_spec = pl.BlockSpec((BKC, BM, wi), lambda i, j, k: (k, i, 0))
    w_spec = pl.BlockSpec((1, BKC * wi, wo), lambda i, j, k: (k, 0, j))
    d_spec = pl.BlockSpec((BM, 128), lambda i, j, k: (i, 0))
    io_spec = pl.BlockSpec((1, BM, wo), lambda i, j, k: (j, i, 0))

    in_specs = [x_spec, w_spec, d_spec]
    args = [x, w4, d2]
    if mode == "mid":
        in_specs += [x_spec,
                     pl.BlockSpec((BKC, 1, wi), lambda i, j, k: (k, 0, 0))]
        args += [hsp, b.reshape(Cin, 1, wi)]
    if init is not None:
        in_specs.append(io_spec)
        args.append(init)

    def body(*refs):
        if mode == "mid" and init is not None:
            x_ref, w_ref, d_ref, h_ref, b_ref, i_ref, o_ref, acc = refs
        elif mode == "mid":
            x_ref, w_ref, d_ref, h_ref, b_ref, o_ref, acc = refs
            i_ref = None
        elif init is not None:
            x_ref, w_ref, d_ref, i_ref, o_ref, acc = refs
        else:
            x_ref, w_ref, d_ref, o_ref, acc = refs
            i_ref = None
        k = pl.program_id(2)

        @pl.when(k == 0)
        def _():
            acc[...] = jnp.zeros((BM, wo), jnp.float32)

        d1 = d_ref[:, :1]
        if mode == "mid":
            xs = [jnp.maximum(d1 * (x_ref[t] + h_ref[t])
                              + b_ref[t, 0][None, :], 0.0)
                  for t in range(BKC)]
            xb = xs[0] if BKC == 1 else jnp.concatenate(xs, axis=1)
        else:
            xb = x_ref[...]
        acc[...] += jnp.dot(xb, w_ref[0], preferred_element_type=jnp.float32)

        @pl.when(k == KG - 1)
        def _():
            r = d1 * acc[...]
            if i_ref is not None:
                r = r + i_ref[0]
            o_ref[0] = r

    return pl.pallas_call(
        body,
        grid=grid,
        in_specs=in_specs,
        out_specs=io_spec,
        out_shape=jax.ShapeDtypeStruct((Cout, NP, wo), jnp.float32),
        scratch_shapes=[pltpu.VMEM((BM, wo), jnp.float32)],
        compiler_params=pltpu.CompilerParams(
            dimension_semantics=("parallel", "parallel", "arbitrary")),
    )(*args)


def _elemwise(body, out_shape, *arrays):
    spec = pl.BlockSpec((BM, 128), lambda i: (i, 0))
    return pl.pallas_call(
        body,
        grid=(NP // BM,),
        in_specs=[spec] * len(arrays),
        out_specs=spec,
        out_shape=out_shape,
    )(*arrays)


def _dinv2(sdeg, mask2):
    def body(s_ref, m_ref, d_ref):
        d_ref[...] = m_ref[...] * lax.rsqrt(s_ref[...] + 1.0)

    return _elemwise(body, jax.ShapeDtypeStruct((NP, 128), jnp.float32),
                     sdeg, mask2)


def _finalize(s, hs, b2, d2):
    bfull = jnp.broadcast_to(b2[None, :], (NP, 128))

    def body(s_ref, h_ref, b_ref, d_ref, o_ref):
        o_ref[...] = d_ref[...] * (s_ref[...] + h_ref[...]) + b_ref[...]

    return _elemwise(body, jax.ShapeDtypeStruct((NP, 128), jnp.float32),
                     s, hs, bfull, d2)


def _pad_w(w, rows, cols):
    return jnp.pad(w, ((0, rows - w.shape[0]), (0, cols - w.shape[1])))


def kernel(vertices, feats1, feats2, feats3, edge_index, params):
    f32 = jnp.float32
    src = edge_index[0].astype(jnp.int32)
    dst = edge_index[1].astype(jnp.int32)
    bucket = dst // RPT
    oh = (bucket[:, None] == jnp.arange(NB_, dtype=jnp.int32)[None, :])
    rank = jnp.cumsum(oh.astype(jnp.int32), axis=0) - oh.astype(jnp.int32)
    rank = jnp.sum(rank * oh, axis=1)
    cnt = jnp.sum(oh, axis=0)
    pos = bucket * CAPE + rank
    src_blk = jnp.full((NB_ * CAPE,), NP - 1, jnp.int32).at[pos].set(src)
    dstl_blk = jnp.zeros((NB_ * CAPE,), jnp.int32).at[pos].set(dst - bucket * RPT)
    sidx = src_blk.reshape(NB_, NBK, GB, EG)
    didx = dstl_blk.reshape(NB_, NBK, GB, EG)
    bcnt = ((cnt + (GB * EG - 1)) // (GB * EG)).astype(jnp.int32)
    bcnt = jnp.broadcast_to(bcnt[:, None], (NB_, 16))

    mask2 = jnp.broadcast_to(
        (jnp.arange(NP) < N).astype(f32)[:, None], (NP, 128))
    sdeg = _prop(mask2[None], sidx, didx, bcnt)[0]
    d2 = _dinv2(sdeg, mask2)

    p1, p2, p3 = params["block1"], params["block2"], params["block3"]

    def chain_rest(hs0, p):
        s0 = _prop(hs0, sidx, didx, bcnt)
        hs1 = _mm(s0, p["W1"], p["b0"], d2, hs0, None, "mid", 256)
        s1 = _prop(hs1, sidx, didx, bcnt)
        hs2 = _mm(s1, p["W2"], p["b1"], d2, hs1, None, "mid", 256)
        s2 = _prop(hs2, sidx, didx, bcnt)
        hs3 = _mm(s2, p["W3"], p["b2"], d2, hs2, None, "mid", 128)
        s3 = _prop(hs3, sidx, didx, bcnt)
        hs4 = _mm(s3, _pad_w(p["W4"], 128, 128), p["b3"], d2, hs3, None,
                  "mid", 128)
        s4 = _prop(hs4, sidx, didx, bcnt)
        b4p = jnp.pad(p["b4"], (0, 128 - 3))
        coord = _finalize(s4[0], hs4[0], b4p, d2)[:N, :3]
        return s3, hs3, coord

    x0 = jnp.concatenate([feats1, vertices], axis=1)
    x0 = jnp.pad(x0, ((0, NP - N), (0, 1536 - 1283)))
    hs0 = _mm(x0, _pad_w(p1["W0"], 1536, 1024), None, d2, None, None,
              "raw", 256)
    s3_1, hs3_1, coord_1 = chain_rest(hs0, p1)

    pinit = _mm(s3_1, p2["W0"][1280:, :], p1["b3"], d2, hs3_1, None,
                "mid", 256)
    f2p = jnp.pad(feats2, ((0, NP - N), (0, 0)))
    hs0 = _mm(f2p, p2["W0"][:1280, :], None, d2, None, pinit, "raw", 256)
    s3_2, hs3_2, coord_2 = chain_rest(hs0, p2)

    pinit = _mm(s3_2, p3["W0"][1280:, :], p2["b3"], d2, hs3_2, None,
                "mid", 256)
    f3p = jnp.pad(feats3, ((0, NP - N), (0, 0)))
    hs0 = _mm(f3p, p3["W0"][:1280, :], None, d2, None, pinit, "raw", 256)
    _, _, coord_3 = chain_rest(hs0, p3)

    return (vertices, coord_1, coord_1, coord_2, coord_2, coord_3)


# speedup vs baseline: 1.2911x; 1.0004x over previous
import functools

import jax
import jax.numpy as jnp
from jax import lax
from jax.experimental import pallas as pl
from jax.experimental.pallas import tpu as pltpu
from jax.experimental.pallas import tpu_sc as plsc

N = 10000
NP = 10240
E = 160000
NTILES = 16
NCORES = 2
NB_ = NTILES * NCORES
RPT = NP // NB_
EG = 128
HG = 64
GB = 8
GCAP = 1256
NBK = GCAP // GB
CAPE = GCAP * EG
BM = 512


@functools.lru_cache(maxsize=None)
def _make_prop(C, W):
    mesh = plsc.VectorSubcoreMesh(core_axis_name="c", subcore_axis_name="s")
    NS = W // 16

    def body(hs, sidx, didx, bcnt, out, acc, ra, rb, iv, dv, bv, sga, sgb):
        cid = lax.axis_index("c")
        sid = lax.axis_index("s")
        tid = cid * NTILES + sid
        zvec = jnp.zeros((16,), jnp.float32)
        bufs = (ra, rb)
        gsems = (sga, sgb)

        pltpu.sync_copy(bcnt, bv)
        myb = bv[tid][0]

        for chunk in range(C):
            def zrow(i, carry):
                for j in range(NS):
                    acc[i, pl.ds(j * 16, 16)] = zvec
                return carry

            lax.fori_loop(0, RPT, zrow, 0)
            off = jnp.full((16,), chunk * NP, jnp.int32)

            def accumulate(buf, g, h):
                lane = lax.iota(jnp.int32, 16)
                zi = jnp.zeros((16,), jnp.int32)

                def jbody(j, cj):
                    dvec = dv[g, pl.ds(h * HG + j * 16, 16)]
                    for k in range(16):
                        rowv = zi + dvec[k]

                        @plsc.parallel_loop(0, NS, unroll=NS)
                        def _(q, k=k):
                            s = pl.ds(q * 16, 16)
                            plsc.addupdate_scatter(
                                acc, [rowv, lane + q * 16],
                                buf[j * 16 + k, s])
                    return cj

                lax.fori_loop(0, HG // 16, jbody, 0)

            def fire(g, h, t):
                return pltpu.async_copy(
                    hs.at[iv.at[g, pl.ds(h * HG, HG)]], bufs[t], gsems[t])

            def wait(g, h, t):
                pltpu.make_async_copy(
                    hs.at[iv.at[g, pl.ds(h * HG, HG)]], bufs[t],
                    gsems[t]).wait()

            def block_body(nb, carry):
                pltpu.sync_copy(sidx.at[tid, nb], iv)
                pltpu.sync_copy(didx.at[tid, nb], dv)
                for r in range(GB):
                    for j in range(8):
                        iv[r, pl.ds(j * 16, 16)] = (
                            iv[r, pl.ds(j * 16, 16)] + off)
                fire(0, 0, 0)

                def gbody(i, c2):
                    for t in range(2):
                        g, h = divmod(i * 2 + t, 2)
                        gn, hn = divmod(i * 2 + t + 1, 2)

                        @pl.when(gn < GB)
                        def _():
                            fire(gn, hn, 1 - t)

                        wait(g, h, t)
                        accumulate(bufs[t], g, h)
                    return c2

                lax.fori_loop(0, GB, gbody, 0)
                return carry

            lax.fori_loop(0, myb, block_body, 0)

            pltpu.sync_copy(
                acc, out.at[pl.ds(chunk * NP + tid * RPT, RPT)])

    return pl.kernel(
        body,
        mesh=mesh,
        compiler_params=pltpu.CompilerParams(needs_layout_passes=False),
        out_type=jax.ShapeDtypeStruct((C * NP, W), jnp.float32),
        scratch_types=[
            pltpu.VMEM((RPT, W), jnp.float32),
            pltpu.VMEM((HG, W), jnp.float32),
            pltpu.VMEM((HG, W), jnp.float32),
            pltpu.VMEM((GB, EG), jnp.int32),
            pltpu.VMEM((GB, EG), jnp.int32),
            pltpu.VMEM((NB_, 16), jnp.int32),
            pltpu.SemaphoreType.DMA,
            pltpu.SemaphoreType.DMA,
        ],
    )


def _prop(hs3, sidx, didx, bcnt):
    C, _, W = hs3.shape
    out = _make_prop(C, W)(hs3.reshape(C * NP, W), sidx, didx, bcnt)
    return out.reshape(C, NP, W)


def _mm(x, w, b, d2, hsp, init, mode, wo):
    if mode == "raw":
        K = x.shape[1]
        wi = 256
    else:
        wi = x.shape[2]
        K = x.shape[0] * wi
    Fout = w.shape[1]
    Cin = K // wi
    BKC = 2 if (Cin % 2 == 0 and wi < 256) else 1
    KG = Cin // BKC
    Cout = Fout // wo
    w4 = w.reshape(KG, BKC * wi, Fout).astype(jnp.bfloat16)

    grid = (NP // BM, Cout, KG)

    if mode == "raw":
        x_spec = pl.BlockSpec((BM, BKC * wi), lambda i, j, k: (i, k))
    else:
        x_spec = pl.BlockSpec((BKC, BM, wi), lambda i, j, k: (k, i, 0))
    w_spec = pl.BlockSpec((1, BKC * wi, wo), lambda i, j, k: (k, 0, j))
    d_spec = pl.BlockSpec((BM, 128), lambda i, j, k: (i, 0))
    io_spec = pl.BlockSpec((1, BM, wo), lambda i, j, k: (j, i, 0))

    in_specs = [x_spec, w_spec, d_spec]
    args = [x, w4, d2]
    if mode == "mid":
        in_specs += [x_spec,
                     pl.BlockSpec((BKC, 1, wi), lambda i, j, k: (k, 0, 0))]
        args += [hsp, b.reshape(Cin, 1, wi)]
    if init is not None:
        in_specs.append(io_spec)
        args.append(init)

    def body(*refs):
        if mode == "mid" and init is not None:
            x_ref, w_ref, d_ref, h_ref, b_ref, i_ref, o_ref, acc = refs
        elif mode == "mid":
            x_ref, w_ref, d_ref, h_ref, b_ref, o_ref, acc = refs
            i_ref = None
        elif init is not None:
            x_ref, w_ref, d_ref, i_ref, o_ref, acc = refs
        else:
            x_ref, w_ref, d_ref, o_ref, acc = refs
            i_ref = None
        k = pl.program_id(2)

        @pl.when(k == 0)
        def _():
            acc[...] = jnp.zeros((BM, wo), jnp.float32)

        d1 = d_ref[:, :1]
        if mode == "mid":
            xs = [jnp.maximum(d1 * (x_ref[t] + h_ref[t])
                              + b_ref[t, 0][None, :], 0.0)
                  for t in range(BKC)]
            xb = xs[0] if BKC == 1 else jnp.concatenate(xs, axis=1)
        else:
            xb = x_ref[...]
        acc[...] += jnp.dot(xb.astype(jnp.bfloat16), w_ref[0],
                            preferred_element_type=jnp.float32)

        @pl.when(k == KG - 1)
        def _():
            r = d1 * acc[...]
            if i_ref is not None:
                r = r + i_ref[0]
            o_ref[0] = r

    return pl.pallas_call(
        body,
        grid=grid,
        in_specs=in_specs,
        out_specs=io_spec,
        out_shape=jax.ShapeDtypeStruct((Cout, NP, wo), jnp.float32),
        scratch_shapes=[pltpu.VMEM((BM, wo), jnp.float32)],
        compiler_params=pltpu.CompilerParams(
            dimension_semantics=("parallel", "parallel", "arbitrary")),
    )(*args)


def _elemwise(body, out_shape, *arrays):
    spec = pl.BlockSpec((BM, 128), lambda i: (i, 0))
    return pl.pallas_call(
        body,
        grid=(NP // BM,),
        in_specs=[spec] * len(arrays),
        out_specs=spec,
        out_shape=out_shape,
    )(*arrays)


def _dinv2(sdeg, mask2):
    def body(s_ref, m_ref, d_ref):
        d_ref[...] = m_ref[...] * lax.rsqrt(s_ref[...] + 1.0)

    return _elemwise(body, jax.ShapeDtypeStruct((NP, 128), jnp.float32),
                     sdeg, mask2)


def _finalize(s, hs, b2, d2):
    bfull = jnp.broadcast_to(b2[None, :], (NP, 128))

    def body(s_ref, h_ref, b_ref, d_ref, o_ref):
        o_ref[...] = d_ref[...] * (s_ref[...] + h_ref[...]) + b_ref[...]

    return _elemwise(body, jax.ShapeDtypeStruct((NP, 128), jnp.float32),
                     s, hs, bfull, d2)


def _pad_w(w, rows, cols):
    return jnp.pad(w, ((0, rows - w.shape[0]), (0, cols - w.shape[1])))


def kernel(vertices, feats1, feats2, feats3, edge_index, params):
    f32 = jnp.float32
    src = edge_index[0].astype(jnp.int32)
    dst = edge_index[1].astype(jnp.int32)
    bucket = dst // RPT
    oh = (bucket[:, None] == jnp.arange(NB_, dtype=jnp.int32)[None, :])
    rank = jnp.cumsum(oh.astype(jnp.int32), axis=0) - oh.astype(jnp.int32)
    rank = jnp.sum(rank * oh, axis=1)
    cnt = jnp.sum(oh, axis=0)
    pos = bucket * CAPE + rank
    src_blk = jnp.full((NB_ * CAPE,), NP - 1, jnp.int32).at[pos].set(src)
    dstl_blk = jnp.zeros((NB_ * CAPE,), jnp.int32).at[pos].set(dst - bucket * RPT)
    sidx = src_blk.reshape(NB_, NBK, GB, EG)
    didx = dstl_blk.reshape(NB_, NBK, GB, EG)
    bcnt = ((cnt + (GB * EG - 1)) // (GB * EG)).astype(jnp.int32)
    bcnt = jnp.broadcast_to(bcnt[:, None], (NB_, 16))

    mask2 = jnp.broadcast_to(
        (jnp.arange(NP) < N).astype(f32)[:, None], (NP, 128))
    sdeg = _prop(mask2[None], sidx, didx, bcnt)[0]
    d2 = _dinv2(sdeg, mask2)

    p1, p2, p3 = params["block1"], params["block2"], params["block3"]

    def chain_rest(hs0, p):
        s0 = _prop(hs0, sidx, didx, bcnt)
        hs1 = _mm(s0, p["W1"], p["b0"], d2, hs0, None, "mid", 256)
        s1 = _prop(hs1, sidx, didx, bcnt)
        hs2 = _mm(s1, p["W2"], p["b1"], d2, hs1, None, "mid", 256)
        s2 = _prop(hs2, sidx, didx, bcnt)
        hs3 = _mm(s2, p["W3"], p["b2"], d2, hs2, None, "mid", 128)
        s3 = _prop(hs3, sidx, didx, bcnt)
        hs4 = _mm(s3, _pad_w(p["W4"], 128, 128), p["b3"], d2, hs3, None,
                  "mid", 128)
        s4 = _prop(hs4, sidx, didx, bcnt)
        b4p = jnp.pad(p["b4"], (0, 128 - 3))
        coord = _finalize(s4[0], hs4[0], b4p, d2)[:N, :3]
        return s3, hs3, coord

    x0 = jnp.concatenate([feats1, vertices], axis=1)
    x0 = jnp.pad(x0, ((0, NP - N), (0, 1536 - 1283)))
    hs0 = _mm(x0, _pad_w(p1["W0"], 1536, 1024), None, d2, None, None,
              "raw", 256)
    s3_1, hs3_1, coord_1 = chain_rest(hs0, p1)

    pinit = _mm(s3_1, p2["W0"][1280:, :], p1["b3"], d2, hs3_1, None,
                "mid", 256)
    f2p = jnp.pad(feats2, ((0, NP - N), (0, 0)))
    hs0 = _mm(f2p, p2["W0"][:1280, :], None, d2, None, pinit, "raw", 256)
    s3_2, hs3_2, coord_2 = chain_rest(hs0, p2)

    pinit = _mm(s3_2, p3["W0"][1280:, :], p2["b3"], d2, hs3_2, None,
                "mid", 256)
    f3p = jnp.pad(feats3, ((0, NP - N), (0, 0)))
    hs0 = _mm(f3p, p3["W0"][:1280, :], None, d2, None, pinit, "raw", 256)
    _, _, coord_3 = chain_rest(hs0, p3)

    return (vertices, coord_1, coord_1, coord_2, coord_2, coord_3)
